# baseline (speedup 1.0000x reference)
import functools

import jax
import jax.numpy as jnp
from jax import lax
from jax.experimental import pallas as pl
from jax.experimental.pallas import tpu as pltpu
from jax.experimental.pallas import tpu_sc as plsc

N = 10000
E = 320000
F1 = 64
NC = 2
NS = 16
NW = NC * NS
EPW = E // NW
CH = 80
NCH = EPW // CH
NP = 10240
STRIPE = NP // NS

_mesh = plsc.VectorSubcoreMesh(core_axis_name="c", subcore_axis_name="s")


@functools.partial(
    pl.kernel,
    out_type=jax.ShapeDtypeStruct((NC, NP), jnp.float32),
    mesh=_mesh,
    scratch_types=[
        pltpu.VMEM((NCH, CH), jnp.int32),
        pltpu.VMEM((CH,), jnp.float32),
        pltpu.VMEM_SHARED((NP,), jnp.float32),
        pltpu.SemaphoreType.DMA,
    ],
)
def _sc_degree(row_hbm, ones_hbm, zeros_hbm, out_hbm, idx_v, ones_v, acc,
               sem):
    cid = lax.axis_index("c")
    sid = lax.axis_index("s")
    wid = sid * NC + cid
    pltpu.sync_copy(row_hbm.at[wid], idx_v)
    pltpu.sync_copy(ones_hbm, ones_v)
    pltpu.sync_copy(zeros_hbm.at[pl.ds(sid * STRIPE, STRIPE)],
                    acc.at[pl.ds(sid * STRIPE, STRIPE)])
    plsc.subcore_barrier()

    def body(c, carry):
        pltpu.async_copy(ones_v, acc.at[idx_v.at[c]], sem, add=True)
        return carry

    lax.fori_loop(0, NCH, body, 0)

    def drain(c, carry):
        pltpu.make_async_copy(ones_v, acc.at[idx_v.at[0]], sem).wait()
        return carry

    lax.fori_loop(0, NCH, drain, 0)
    plsc.subcore_barrier()
    pltpu.sync_copy(acc.at[pl.ds(sid * STRIPE, STRIPE)],
                    out_hbm.at[cid, pl.ds(sid * STRIPE, STRIPE)])


@functools.partial(
    pl.kernel,
    out_type=jax.ShapeDtypeStruct((NC, NP, F1), jnp.float32),
    mesh=_mesh,
    scratch_types=[
        pltpu.VMEM((NCH, CH), jnp.int32),
        pltpu.VMEM((NCH, CH), jnp.int32),
        [pltpu.VMEM((CH, F1), jnp.float32)] * 4,
        pltpu.VMEM_SHARED((NP, F1), jnp.float32),
        pltpu.VMEM_SHARED((NP, F1), jnp.float32),
        [pltpu.SemaphoreType.DMA] * 4,
        [pltpu.SemaphoreType.DMA] * 4,
    ],
    compiler_params=pltpu.CompilerParams(use_tc_tiling_on_sc=False),
)
def _sc_edge_pass(z_hbm, row_hbm, col_hbm, zeros_hbm, out_hbm,
                  rows_v, cols_v, bufs, acc, zbuf, gsems, ssems):
    cid = lax.axis_index("c")
    sid = lax.axis_index("s")
    wid = sid * NC + cid
    pltpu.sync_copy(row_hbm.at[wid], rows_v)
    pltpu.sync_copy(col_hbm.at[wid], cols_v)
    pltpu.sync_copy(zeros_hbm.at[pl.ds(sid * STRIPE, STRIPE)],
                    acc.at[pl.ds(sid * STRIPE, STRIPE)])

    @pl.when(sid < NS - 1)
    def _():
        pltpu.sync_copy(z_hbm.at[pl.ds(sid * STRIPE, STRIPE)],
                        zbuf.at[pl.ds(sid * STRIPE, STRIPE)])

    @pl.when(sid == NS - 1)
    def _():
        pltpu.sync_copy(z_hbm.at[pl.ds((NS - 1) * STRIPE, N - (NS - 1) * STRIPE)],
                        zbuf.at[pl.ds((NS - 1) * STRIPE, N - (NS - 1) * STRIPE)])
    plsc.subcore_barrier()

    def g(c, b, m4=0):
        src = z_hbm if m4 == 1 else zbuf
        pltpu.async_copy(src.at[rows_v.at[c]], bufs[b], gsems[b])

    def gw(c, b, m4=0):
        src = z_hbm if m4 == 1 else zbuf
        pltpu.make_async_copy(src.at[rows_v.at[c]], bufs[b],
                              gsems[b]).wait()

    def s(c, b):
        pltpu.async_copy(bufs[b], acc.at[cols_v.at[c]], ssems[b], add=True)

    def sw(c, b):
        pltpu.make_async_copy(bufs[b], acc.at[cols_v.at[c]],
                              ssems[b]).wait()

    g(0, 0, 0)
    g(1, 1, 1)
    gw(0, 0, 0); s(0, 0); g(2, 2, 2)
    gw(1, 1, 1); s(1, 1); g(3, 3, 3)

    def quad(k, carry):
        c0 = 2 + 4 * k
        for j in range(4):
            c = c0 + j
            b = (2 + j) % 4
            b2 = (b + 2) % 4
            m4 = (2 + j) % 4
            gw(c, b, m4)
            s(c, b)
            sw(c - 2, b2)
            g(c + 2, b2, (m4 + 2) % 4)
        return carry

    lax.fori_loop(0, (NCH - 5) // 4, quad, 0)
    gw(NCH - 3, 2, (NCH - 3) % 4); s(NCH - 3, 2); sw(NCH - 5, 0)
    g(NCH - 1, 0, (NCH - 1) % 4)
    gw(NCH - 2, 3, (NCH - 2) % 4); s(NCH - 2, 3); sw(NCH - 4, 1)
    gw(NCH - 1, 0, (NCH - 1) % 4); s(NCH - 1, 0)
    sw(NCH - 3, 2)
    sw(NCH - 2, 3)
    sw(NCH - 1, 0)
    plsc.subcore_barrier()
    pltpu.sync_copy(acc.at[pl.ds(sid * STRIPE, STRIPE)],
                    out_hbm.at[cid, pl.ds(sid * STRIPE, STRIPE)])


def _mm2_body(x_ref, w0_ref, w1_ref, m0_ref, m1_ref):
    x = x_ref[...]
    m0_ref[...] = jnp.dot(x, w0_ref[...], preferred_element_type=jnp.float32)
    m1_ref[...] = jnp.dot(x, w1_ref[...], preferred_element_type=jnp.float32)


def _scale_body(p0_ref, p1_ref, m1_ref, dinv_ref, z1_ref):
    deg = p0_ref[...].reshape(N, 1) + p1_ref[...].reshape(N, 1)
    dinv = jnp.where(deg > 0.0, lax.rsqrt(jnp.maximum(deg, 1.0e-12)), 0.0)
    dinv_ref[...] = dinv
    z1_ref[...] = m1_ref[...] * dinv


def _mid_body(m0_ref, b_ref, dinv_ref, a0_ref, a1_ref, h_ref, z2_ref):
    dinv = dinv_ref[...]
    s = -dinv * (a0_ref[...].reshape(N, F1) + a1_ref[...].reshape(N, F1))
    h = jnp.maximum(m0_ref[...] + s + b_ref[...], 0.0)
    h_ref[...] = h
    z2_ref[...] = dinv * h


def _hw_body(h_ref, w_ref, hw_ref):
    hw_ref[...] = jnp.dot(h_ref[...], w_ref[...],
                          preferred_element_type=jnp.float32)


def _final_body(hw_ref, w1_ref, b_ref, dinv_ref, a0_ref, a1_ref, o_ref):
    s = -dinv_ref[...] * (a0_ref[...].reshape(N, F1)
                          + a1_ref[...].reshape(N, F1))
    o_ref[...] = (hw_ref[...]
                  + jnp.dot(s, w1_ref[...], preferred_element_type=jnp.float32)
                  + b_ref[...])


def _full(shape):
    return pl.BlockSpec(shape, lambda i: tuple(0 for _ in shape))


def _part(k, shape):
    return pl.BlockSpec((1,) + shape, lambda i: (k,) + tuple(0 for _ in shape))


def kernel(x, edge_index, W0a, W1a, ba, W0b, W1b, bb):
    row = edge_index[0].reshape(NW, NCH, CH)
    col = edge_index[1].reshape(NW, NCH, CH)
    ones_ch = jnp.ones((CH,), jnp.float32)
    zeros1 = jnp.zeros((NP,), jnp.float32)
    zeros2 = jnp.zeros((NP, F1), jnp.float32)

    degp = _sc_degree(row, ones_ch, zeros1).reshape(NC, NP, 1)

    m0, m1 = pl.pallas_call(
        _mm2_body,
        grid=(1,),
        in_specs=[_full((N, 128)), _full((128, F1)), _full((128, F1))],
        out_specs=[_full((N, F1)), _full((N, F1))],
        out_shape=[jax.ShapeDtypeStruct((N, F1), jnp.float32),
                   jax.ShapeDtypeStruct((N, F1), jnp.float32)],
    )(x, W0a, W1a)

    dinv, z1 = pl.pallas_call(
        _scale_body,
        grid=(1,),
        in_specs=[_part(0, (N, 1)), _part(1, (N, 1)), _full((N, F1))],
        out_specs=[_full((N, 1)), _full((N, F1))],
        out_shape=[jax.ShapeDtypeStruct((N, 1), jnp.float32),
                   jax.ShapeDtypeStruct((N, F1), jnp.float32)],
    )(degp, degp, m1)

    acc1 = _sc_edge_pass(z1, row, col, zeros2)

    h, z2 = pl.pallas_call(
        _mid_body,
        grid=(1,),
        in_specs=[_full((N, F1)), _full((1, F1)), _full((N, 1)),
                  _part(0, (N, F1)), _part(1, (N, F1))],
        out_specs=[_full((N, F1)), _full((N, F1))],
        out_shape=[jax.ShapeDtypeStruct((N, F1), jnp.float32),
                   jax.ShapeDtypeStruct((N, F1), jnp.float32)],
    )(m0, ba.reshape(1, F1), dinv, acc1, acc1)

    hw = pl.pallas_call(
        _hw_body,
        grid=(1,),
        in_specs=[_full((N, F1)), _full((F1, 128))],
        out_specs=_full((N, 128)),
        out_shape=jax.ShapeDtypeStruct((N, 128), jnp.float32),
    )(h, W0b)

    acc2 = _sc_edge_pass(z2, row, col, zeros2)

    out = pl.pallas_call(
        _final_body,
        grid=(1,),
        in_specs=[_full((N, 128)), _full((F1, 128)), _full((1, 128)),
                  _full((N, 1)), _part(0, (N, F1)), _part(1, (N, F1))],
        out_specs=_full((N, 128)),
        out_shape=jax.ShapeDtypeStruct((N, 128), jnp.float32),
    )(hw, W1b, bb.reshape(1, 128), dinv, acc2, acc2)

    return out


# speedup vs baseline: 1.1645x; 1.1645x over previous
import functools

import jax
import jax.numpy as jnp
from jax import lax
from jax.experimental import pallas as pl
from jax.experimental.pallas import tpu as pltpu
from jax.experimental.pallas import tpu_sc as plsc

N = 10000
E = 320000
F1 = 64
NC = 2
NS = 16
NW = NC * NS
EPW = E // NW
CH = 80
NCH = EPW // CH
NP = 10240
STRIPE = NP // NS

_mesh = plsc.VectorSubcoreMesh(core_axis_name="c", subcore_axis_name="s")


@functools.partial(
    pl.kernel,
    out_type=jax.ShapeDtypeStruct((NC, NP), jnp.float32),
    mesh=_mesh,
    scratch_types=[
        pltpu.VMEM((NCH, CH), jnp.int32),
        pltpu.VMEM((CH,), jnp.float32),
        pltpu.VMEM_SHARED((NP,), jnp.float32),
        pltpu.SemaphoreType.DMA,
    ],
)
def _sc_degree(row_hbm, ones_hbm, zeros_hbm, out_hbm, idx_v, ones_v, acc,
               sem):
    cid = lax.axis_index("c")
    sid = lax.axis_index("s")
    wid = sid * NC + cid
    pltpu.sync_copy(row_hbm.at[wid], idx_v)
    pltpu.sync_copy(ones_hbm, ones_v)
    pltpu.sync_copy(zeros_hbm.at[pl.ds(sid * STRIPE, STRIPE)],
                    acc.at[pl.ds(sid * STRIPE, STRIPE)])
    plsc.subcore_barrier()

    def body(c, carry):
        pltpu.async_copy(ones_v, acc.at[idx_v.at[c]], sem, add=True)
        return carry

    lax.fori_loop(0, NCH, body, 0)

    def drain(c, carry):
        pltpu.make_async_copy(ones_v, acc.at[idx_v.at[0]], sem).wait()
        return carry

    lax.fori_loop(0, NCH, drain, 0)
    plsc.subcore_barrier()
    pltpu.sync_copy(acc.at[pl.ds(sid * STRIPE, STRIPE)],
                    out_hbm.at[cid, pl.ds(sid * STRIPE, STRIPE)])


@functools.partial(
    pl.kernel,
    out_type=jax.ShapeDtypeStruct((NC, NP, F1), jnp.float32),
    mesh=_mesh,
    scratch_types=[
        pltpu.VMEM((NCH, CH), jnp.int32),
        pltpu.VMEM((NCH, CH), jnp.int32),
        [pltpu.VMEM((CH, F1), jnp.float32)] * 4,
        pltpu.VMEM_SHARED((NP, F1), jnp.float32),
        pltpu.VMEM_SHARED((NP, F1), jnp.float32),
        [pltpu.SemaphoreType.DMA] * 4,
        [pltpu.SemaphoreType.DMA] * 4,
    ],
    compiler_params=pltpu.CompilerParams(use_tc_tiling_on_sc=False),
)
def _sc_edge_pass(z_hbm, row_hbm, col_hbm, zeros_hbm, out_hbm,
                  rows_v, cols_v, bufs, acc, zbuf, gsems, ssems):
    cid = lax.axis_index("c")
    sid = lax.axis_index("s")
    wid = sid * NC + cid
    pltpu.sync_copy(row_hbm.at[wid], rows_v)
    pltpu.sync_copy(col_hbm.at[wid], cols_v)
    pltpu.sync_copy(zeros_hbm.at[pl.ds(sid * STRIPE, STRIPE)],
                    acc.at[pl.ds(sid * STRIPE, STRIPE)])

    @pl.when(sid < NS - 1)
    def _():
        pltpu.sync_copy(z_hbm.at[pl.ds(sid * STRIPE, STRIPE)],
                        zbuf.at[pl.ds(sid * STRIPE, STRIPE)])

    @pl.when(sid == NS - 1)
    def _():
        pltpu.sync_copy(z_hbm.at[pl.ds((NS - 1) * STRIPE, N - (NS - 1) * STRIPE)],
                        zbuf.at[pl.ds((NS - 1) * STRIPE, N - (NS - 1) * STRIPE)])
    plsc.subcore_barrier()

    def g(c, b):
        pltpu.async_copy(zbuf.at[rows_v.at[c]], bufs[b], gsems[b])

    def gw(c, b):
        pltpu.make_async_copy(zbuf.at[rows_v.at[c]], bufs[b],
                              gsems[b]).wait()

    def s(c, b):
        pltpu.async_copy(bufs[b], acc.at[cols_v.at[c]], ssems[b], add=True)

    def sw(c, b):
        pltpu.make_async_copy(bufs[b], acc.at[cols_v.at[c]],
                              ssems[b]).wait()

    g(0, 0)
    g(1, 1)
    gw(0, 0); s(0, 0); g(2, 2)
    gw(1, 1); s(1, 1); g(3, 3)

    def quad(k, carry):
        c0 = 2 + 4 * k
        for j in range(4):
            c = c0 + j
            b = (2 + j) % 4
            b2 = (b + 2) % 4
            gw(c, b)
            s(c, b)
            sw(c - 2, b2)
            g(c + 2, b2)
        return carry

    lax.fori_loop(0, (NCH - 5) // 4, quad, 0)
    gw(NCH - 3, 2); s(NCH - 3, 2); sw(NCH - 5, 0); g(NCH - 1, 0)
    gw(NCH - 2, 3); s(NCH - 2, 3); sw(NCH - 4, 1)
    gw(NCH - 1, 0); s(NCH - 1, 0)
    sw(NCH - 3, 2)
    sw(NCH - 2, 3)
    sw(NCH - 1, 0)
    plsc.subcore_barrier()
    pltpu.sync_copy(acc.at[pl.ds(sid * STRIPE, STRIPE)],
                    out_hbm.at[cid, pl.ds(sid * STRIPE, STRIPE)])


def _mm2_body(x_ref, w0_ref, w1_ref, m0_ref, m1_ref):
    x = x_ref[...]
    m0_ref[...] = jnp.dot(x, w0_ref[...], preferred_element_type=jnp.float32)
    m1_ref[...] = jnp.dot(x, w1_ref[...], preferred_element_type=jnp.float32)


def _scale_body(p0_ref, p1_ref, m1_ref, dinv_ref, z1_ref):
    deg = p0_ref[...].reshape(N, 1) + p1_ref[...].reshape(N, 1)
    dinv = jnp.where(deg > 0.0, lax.rsqrt(jnp.maximum(deg, 1.0e-12)), 0.0)
    dinv_ref[...] = dinv
    z1_ref[...] = m1_ref[...] * dinv


def _mid_body(m0_ref, b_ref, dinv_ref, a0_ref, a1_ref, h_ref, z2_ref):
    dinv = dinv_ref[...]
    s = -dinv * (a0_ref[...].reshape(N, F1) + a1_ref[...].reshape(N, F1))
    h = jnp.maximum(m0_ref[...] + s + b_ref[...], 0.0)
    h_ref[...] = h
    z2_ref[...] = dinv * h


def _hw_body(h_ref, w_ref, hw_ref):
    hw_ref[...] = jnp.dot(h_ref[...], w_ref[...],
                          preferred_element_type=jnp.float32)


def _final_body(hw_ref, w1_ref, b_ref, dinv_ref, a0_ref, a1_ref, o_ref):
    s = -dinv_ref[...] * (a0_ref[...].reshape(N, F1)
                          + a1_ref[...].reshape(N, F1))
    o_ref[...] = (hw_ref[...]
                  + jnp.dot(s, w1_ref[...], preferred_element_type=jnp.float32)
                  + b_ref[...])


def _full(shape):
    return pl.BlockSpec(shape, lambda i: tuple(0 for _ in shape))


def _part(k, shape):
    return pl.BlockSpec((1,) + shape, lambda i: (k,) + tuple(0 for _ in shape))


def kernel(x, edge_index, W0a, W1a, ba, W0b, W1b, bb):
    row = edge_index[0].reshape(NW, NCH, CH)
    col = edge_index[1].reshape(NW, NCH, CH)
    ones_ch = jnp.ones((CH,), jnp.float32)
    zeros1 = jnp.zeros((NP,), jnp.float32)
    zeros2 = jnp.zeros((NP, F1), jnp.float32)

    degp = _sc_degree(row, ones_ch, zeros1).reshape(NC, NP, 1)

    m0, m1 = pl.pallas_call(
        _mm2_body,
        grid=(1,),
        in_specs=[_full((N, 128)), _full((128, F1)), _full((128, F1))],
        out_specs=[_full((N, F1)), _full((N, F1))],
        out_shape=[jax.ShapeDtypeStruct((N, F1), jnp.float32),
                   jax.ShapeDtypeStruct((N, F1), jnp.float32)],
    )(x, W0a, W1a)

    dinv, z1 = pl.pallas_call(
        _scale_body,
        grid=(1,),
        in_specs=[_part(0, (N, 1)), _part(1, (N, 1)), _full((N, F1))],
        out_specs=[_full((N, 1)), _full((N, F1))],
        out_shape=[jax.ShapeDtypeStruct((N, 1), jnp.float32),
                   jax.ShapeDtypeStruct((N, F1), jnp.float32)],
    )(degp, degp, m1)

    acc1 = _sc_edge_pass(z1, row, col, zeros2)

    h, z2 = pl.pallas_call(
        _mid_body,
        grid=(1,),
        in_specs=[_full((N, F1)), _full((1, F1)), _full((N, 1)),
                  _part(0, (N, F1)), _part(1, (N, F1))],
        out_specs=[_full((N, F1)), _full((N, F1))],
        out_shape=[jax.ShapeDtypeStruct((N, F1), jnp.float32),
                   jax.ShapeDtypeStruct((N, F1), jnp.float32)],
    )(m0, ba.reshape(1, F1), dinv, acc1, acc1)

    hw = pl.pallas_call(
        _hw_body,
        grid=(1,),
        in_specs=[_full((N, F1)), _full((F1, 128))],
        out_specs=_full((N, 128)),
        out_shape=jax.ShapeDtypeStruct((N, 128), jnp.float32),
    )(h, W0b)

    acc2 = _sc_edge_pass(z2, row, col, zeros2)

    out = pl.pallas_call(
        _final_body,
        grid=(1,),
        in_specs=[_full((N, 128)), _full((F1, 128)), _full((1, 128)),
                  _full((N, 1)), _part(0, (N, F1)), _part(1, (N, F1))],
        out_specs=_full((N, 128)),
        out_shape=jax.ShapeDtypeStruct((N, 128), jnp.float32),
    )(hw, W1b, bb.reshape(1, 128), dinv, acc2, acc2)

    return out


# speedup vs baseline: 1.2204x; 1.0480x over previous
import functools

import jax
import jax.numpy as jnp
from jax import lax
from jax.experimental import pallas as pl
from jax.experimental.pallas import tpu as pltpu
from jax.experimental.pallas import tpu_sc as plsc

N = 10000
E = 320000
F1 = 64
NC = 2
NS = 16
NW = NC * NS
EPW = E // NW
CH = 80
NCH = EPW // CH
NP = 10240
STRIPE = NP // NS

_mesh = plsc.VectorSubcoreMesh(core_axis_name="c", subcore_axis_name="s")


@functools.partial(
    pl.kernel,
    out_type=jax.ShapeDtypeStruct((NC, NP), jnp.float32),
    mesh=_mesh,
    scratch_types=[
        pltpu.VMEM((NCH, CH), jnp.int32),
        pltpu.VMEM((CH,), jnp.float32),
        pltpu.VMEM_SHARED((NP,), jnp.float32),
        pltpu.SemaphoreType.DMA,
    ],
    compiler_params=pltpu.CompilerParams(use_tc_tiling_on_sc=False),
)
def _sc_degree(row_hbm, ones_hbm, zeros_hbm, out_hbm, idx_v, ones_v, acc,
               sem):
    cid = lax.axis_index("c")
    sid = lax.axis_index("s")
    wid = sid * NC + cid
    pltpu.sync_copy(row_hbm.at[wid], idx_v)
    pltpu.sync_copy(ones_hbm, ones_v)
    pltpu.sync_copy(zeros_hbm.at[pl.ds(sid * STRIPE, STRIPE)],
                    acc.at[pl.ds(sid * STRIPE, STRIPE)])
    plsc.subcore_barrier()

    def body(c, carry):
        pltpu.async_copy(ones_v, acc.at[idx_v.at[c]], sem, add=True)
        return carry

    lax.fori_loop(0, NCH, body, 0)

    def drain(c, carry):
        pltpu.make_async_copy(ones_v, acc.at[idx_v.at[0]], sem).wait()
        return carry

    lax.fori_loop(0, NCH, drain, 0)
    plsc.subcore_barrier()
    pltpu.sync_copy(acc.at[pl.ds(sid * STRIPE, STRIPE)],
                    out_hbm.at[cid, pl.ds(sid * STRIPE, STRIPE)])


@functools.partial(
    pl.kernel,
    out_type=jax.ShapeDtypeStruct((NC, NP, F1), jnp.float32),
    mesh=_mesh,
    scratch_types=[
        pltpu.VMEM((NCH, CH), jnp.int32),
        pltpu.VMEM((NCH, CH), jnp.int32),
        [pltpu.VMEM((CH, F1), jnp.float32)] * 4,
        pltpu.VMEM_SHARED((NP, F1), jnp.float32),
        pltpu.VMEM_SHARED((NP, F1), jnp.float32),
        [pltpu.SemaphoreType.DMA] * 4,
        [pltpu.SemaphoreType.DMA] * 4,
    ],
    compiler_params=pltpu.CompilerParams(use_tc_tiling_on_sc=False),
)
def _sc_edge_pass(z_hbm, row_hbm, col_hbm, zeros_hbm, out_hbm,
                  rows_v, cols_v, bufs, acc, zbuf, gsems, ssems):
    cid = lax.axis_index("c")
    sid = lax.axis_index("s")
    wid = sid * NC + cid
    pltpu.sync_copy(row_hbm.at[wid], rows_v)
    pltpu.sync_copy(col_hbm.at[wid], cols_v)
    pltpu.sync_copy(zeros_hbm.at[pl.ds(sid * STRIPE, STRIPE)],
                    acc.at[pl.ds(sid * STRIPE, STRIPE)])

    @pl.when(sid < NS - 1)
    def _():
        pltpu.sync_copy(z_hbm.at[pl.ds(sid * STRIPE, STRIPE)],
                        zbuf.at[pl.ds(sid * STRIPE, STRIPE)])

    @pl.when(sid == NS - 1)
    def _():
        pltpu.sync_copy(z_hbm.at[pl.ds((NS - 1) * STRIPE, N - (NS - 1) * STRIPE)],
                        zbuf.at[pl.ds((NS - 1) * STRIPE, N - (NS - 1) * STRIPE)])
    plsc.subcore_barrier()

    def g(c, b):
        pltpu.async_copy(zbuf.at[rows_v.at[c]], bufs[b], gsems[b])

    def gw(c, b):
        pltpu.make_async_copy(zbuf.at[rows_v.at[c]], bufs[b],
                              gsems[b]).wait()

    def s(c, b):
        pltpu.async_copy(bufs[b], acc.at[cols_v.at[c]], ssems[b], add=True)

    def sw(c, b):
        pltpu.make_async_copy(bufs[b], acc.at[cols_v.at[c]],
                              ssems[b]).wait()

    g(0, 0)
    g(1, 1)
    gw(0, 0); s(0, 0); g(2, 2)
    gw(1, 1); s(1, 1); g(3, 3)

    def quad(k, carry):
        c0 = 2 + 4 * k
        for j in range(4):
            c = c0 + j
            b = (2 + j) % 4
            b2 = (b + 2) % 4
            gw(c, b)
            s(c, b)
            sw(c - 2, b2)
            g(c + 2, b2)
        return carry

    lax.fori_loop(0, (NCH - 5) // 4, quad, 0)
    gw(NCH - 3, 2); s(NCH - 3, 2); sw(NCH - 5, 0); g(NCH - 1, 0)
    gw(NCH - 2, 3); s(NCH - 2, 3); sw(NCH - 4, 1)
    gw(NCH - 1, 0); s(NCH - 1, 0)
    sw(NCH - 3, 2)
    sw(NCH - 2, 3)
    sw(NCH - 1, 0)
    plsc.subcore_barrier()
    pltpu.sync_copy(acc.at[pl.ds(sid * STRIPE, STRIPE)],
                    out_hbm.at[cid, pl.ds(sid * STRIPE, STRIPE)])


def _mm2_body(x_ref, w0_ref, w1_ref, m0_ref, m1_ref):
    x = x_ref[...]
    m0_ref[...] = jnp.dot(x, w0_ref[...], preferred_element_type=jnp.float32)
    m1_ref[...] = jnp.dot(x, w1_ref[...], preferred_element_type=jnp.float32)


def _scale_body(p_ref, m1_ref, dinv_ref, z1_ref):
    p = p_ref[...]
    deg = p[0:1, :] + p[1:2, :]
    dinv_lane = jnp.where(deg > 0.0, lax.rsqrt(jnp.maximum(deg, 1.0e-12)),
                          0.0)
    dinv = jnp.transpose(dinv_lane, (1, 0))[:N]
    dinv_ref[...] = dinv
    z1_ref[...] = m1_ref[...] * dinv


def _mid_body(m0_ref, b_ref, dinv_ref, a0_ref, a1_ref, h_ref, z2_ref):
    dinv = dinv_ref[...]
    s = -dinv * (a0_ref[...].reshape(N, F1) + a1_ref[...].reshape(N, F1))
    h = jnp.maximum(m0_ref[...] + s + b_ref[...], 0.0)
    h_ref[...] = h
    z2_ref[...] = dinv * h


def _hw_body(h_ref, w_ref, hw_ref):
    hw_ref[...] = jnp.dot(h_ref[...], w_ref[...],
                          preferred_element_type=jnp.float32)


def _final_body(hw_ref, w1_ref, b_ref, dinv_ref, a0_ref, a1_ref, o_ref):
    s = -dinv_ref[...] * (a0_ref[...].reshape(N, F1)
                          + a1_ref[...].reshape(N, F1))
    o_ref[...] = (hw_ref[...]
                  + jnp.dot(s, w1_ref[...], preferred_element_type=jnp.float32)
                  + b_ref[...])


def _full(shape):
    return pl.BlockSpec(shape, lambda i: tuple(0 for _ in shape))


def _part(k, shape):
    return pl.BlockSpec((1,) + shape, lambda i: (k,) + tuple(0 for _ in shape))


def kernel(x, edge_index, W0a, W1a, ba, W0b, W1b, bb):
    row = edge_index[0].reshape(NW, NCH, CH)
    col = edge_index[1].reshape(NW, NCH, CH)
    ones_ch = jnp.ones((CH,), jnp.float32)
    zeros1 = jnp.zeros((NP,), jnp.float32)
    zeros2 = jnp.zeros((NP, F1), jnp.float32)

    degp = _sc_degree(row, ones_ch, zeros1)

    m0, m1 = pl.pallas_call(
        _mm2_body,
        grid=(1,),
        in_specs=[_full((N, 128)), _full((128, F1)), _full((128, F1))],
        out_specs=[_full((N, F1)), _full((N, F1))],
        out_shape=[jax.ShapeDtypeStruct((N, F1), jnp.float32),
                   jax.ShapeDtypeStruct((N, F1), jnp.float32)],
    )(x, W0a, W1a)

    dinv, z1 = pl.pallas_call(
        _scale_body,
        grid=(1,),
        in_specs=[_full((NC, NP)), _full((N, F1))],
        out_specs=[_full((N, 1)), _full((N, F1))],
        out_shape=[jax.ShapeDtypeStruct((N, 1), jnp.float32),
                   jax.ShapeDtypeStruct((N, F1), jnp.float32)],
    )(degp, m1)

    acc1 = _sc_edge_pass(z1, row, col, zeros2)

    h, z2 = pl.pallas_call(
        _mid_body,
        grid=(1,),
        in_specs=[_full((N, F1)), _full((1, F1)), _full((N, 1)),
                  _part(0, (N, F1)), _part(1, (N, F1))],
        out_specs=[_full((N, F1)), _full((N, F1))],
        out_shape=[jax.ShapeDtypeStruct((N, F1), jnp.float32),
                   jax.ShapeDtypeStruct((N, F1), jnp.float32)],
    )(m0, ba.reshape(1, F1), dinv, acc1, acc1)

    hw = pl.pallas_call(
        _hw_body,
        grid=(1,),
        in_specs=[_full((N, F1)), _full((F1, 128))],
        out_specs=_full((N, 128)),
        out_shape=jax.ShapeDtypeStruct((N, 128), jnp.float32),
    )(h, W0b)

    acc2 = _sc_edge_pass(z2, row, col, zeros2)

    out = pl.pallas_call(
        _final_body,
        grid=(1,),
        in_specs=[_full((N, 128)), _full((F1, 128)), _full((1, 128)),
                  _full((N, 1)), _part(0, (N, F1)), _part(1, (N, F1))],
        out_specs=_full((N, 128)),
        out_shape=jax.ShapeDtypeStruct((N, 128), jnp.float32),
    )(hw, W1b, bb.reshape(1, 128), dinv, acc2, acc2)

    return out
